# baseline (speedup 1.0000x reference)
import functools

import jax
import jax.numpy as jnp
from jax import lax
from jax.experimental import pallas as pl
from jax.experimental.pallas import tpu as pltpu
from jax.experimental.pallas import tpu_sc as plsc

_N = 10000
_E = 320000
_E2 = _E // 2
_D = 128
_DE = 64

_BN = 2000
_BE2 = 3200

_NC = 2
_NS = 16
_NW = _NC * _NS
_EPW = _E // _NW
_CH = 80
_CH2 = _CH // 2
_NCH = _EPW // _CH
_E2PW = _E2 // _NW
_NPS = 624
_ZR = 104
_TAIL = _N - _NPS * _NS


def _encode_nodes_body(x_ref, wne_ref, bne_ref, ws0_ref, wd0_ref, wn0_ref,
                       bn_ref, nf0_ref, a0s_ref, a0d_ref, cn_ref):
    nf0 = jnp.dot(x_ref[...], wne_ref[...],
                  preferred_element_type=jnp.float32) + bne_ref[...]
    nf0_ref[...] = nf0
    a0s_ref[...] = jnp.dot(nf0, ws0_ref[...], preferred_element_type=jnp.float32)
    a0d_ref[...] = jnp.dot(nf0, wd0_ref[...], preferred_element_type=jnp.float32)
    cn_ref[...] = jnp.dot(nf0, wn0_ref[...],
                          preferred_element_type=jnp.float32) + bn_ref[...]


_encode_nodes = pl.pallas_call(
    _encode_nodes_body,
    grid=(_N // _BN,),
    in_specs=[
        pl.BlockSpec((_BN, _D), lambda i: (i, 0)),
        pl.BlockSpec((_D, _D), lambda i: (0, 0)),
        pl.BlockSpec((1, _D), lambda i: (0, 0)),
        pl.BlockSpec((_D, _DE), lambda i: (0, 0)),
        pl.BlockSpec((_D, _DE), lambda i: (0, 0)),
        pl.BlockSpec((_D, _D), lambda i: (0, 0)),
        pl.BlockSpec((1, _D), lambda i: (0, 0)),
    ],
    out_specs=[
        pl.BlockSpec((_BN, _D), lambda i: (i, 0)),
        pl.BlockSpec((_BN, _DE), lambda i: (i, 0)),
        pl.BlockSpec((_BN, _DE), lambda i: (i, 0)),
        pl.BlockSpec((_BN, _D), lambda i: (i, 0)),
    ],
    out_shape=[
        jax.ShapeDtypeStruct((_N, _D), jnp.float32),
        jax.ShapeDtypeStruct((_N, _DE), jnp.float32),
        jax.ShapeDtypeStruct((_N, _DE), jnp.float32),
        jax.ShapeDtypeStruct((_N, _D), jnp.float32),
    ],
)


def _encode_edges_body(eate_ref, eato_ref, wee_ref, bee_ref, we0_ref, be_ref,
                       ef0_ref, t0_ref):
    dn = (((0,), (0,)), ((), ()))
    y0 = lax.dot_general(eate_ref[...], wee_ref[...], dn,
                         preferred_element_type=jnp.float32) + bee_ref[...]
    y1 = lax.dot_general(eato_ref[...], wee_ref[...], dn,
                         preferred_element_type=jnp.float32) + bee_ref[...]
    ef0 = jnp.concatenate([y0, y1], axis=1)
    ef0_ref[...] = ef0
    t0_ref[...] = jnp.dot(ef0, we0_ref[...],
                          preferred_element_type=jnp.float32) + be_ref[...]


_encode_edges = pl.pallas_call(
    _encode_edges_body,
    grid=(_E2 // _BE2,),
    in_specs=[
        pl.BlockSpec((16, _BE2), lambda i: (0, i)),
        pl.BlockSpec((16, _BE2), lambda i: (0, i)),
        pl.BlockSpec((16, _DE), lambda i: (0, 0)),
        pl.BlockSpec((1, _DE), lambda i: (0, 0)),
        pl.BlockSpec((_D, _D), lambda i: (0, 0)),
        pl.BlockSpec((1, _D), lambda i: (0, 0)),
    ],
    out_specs=[
        pl.BlockSpec((_BE2, _D), lambda i: (i, 0)),
        pl.BlockSpec((_BE2, _D), lambda i: (i, 0)),
    ],
    out_shape=[
        jax.ShapeDtypeStruct((_E2, _D), jnp.float32),
        jax.ShapeDtypeStruct((_E2, _D), jnp.float32),
    ],
)


def _proj_body(nf_ref, a0s_ref, a0d_ref, ws1_ref, wd1_ref, as_ref, ad_ref):
    nf = nf_ref[...]
    as_ref[...] = a0s_ref[...] + jnp.dot(nf, ws1_ref[...],
                                         preferred_element_type=jnp.float32)
    ad_ref[...] = a0d_ref[...] + jnp.dot(nf, wd1_ref[...],
                                         preferred_element_type=jnp.float32)


_proj = pl.pallas_call(
    _proj_body,
    grid=(_N // _BN,),
    in_specs=[
        pl.BlockSpec((_BN, _D), lambda i: (i, 0)),
        pl.BlockSpec((_BN, _DE), lambda i: (i, 0)),
        pl.BlockSpec((_BN, _DE), lambda i: (i, 0)),
        pl.BlockSpec((_D, _DE), lambda i: (0, 0)),
        pl.BlockSpec((_D, _DE), lambda i: (0, 0)),
    ],
    out_specs=[
        pl.BlockSpec((_BN, _DE), lambda i: (i, 0)),
        pl.BlockSpec((_BN, _DE), lambda i: (i, 0)),
    ],
    out_shape=[
        jax.ShapeDtypeStruct((_N, _DE), jnp.float32),
        jax.ShapeDtypeStruct((_N, _DE), jnp.float32),
    ],
)


def _edge_t_body(t0_ref, ef_ref, we1_ref, t_ref):
    t_ref[...] = t0_ref[...] + jnp.dot(ef_ref[...], we1_ref[...],
                                       preferred_element_type=jnp.float32)


_edge_t = pl.pallas_call(
    _edge_t_body,
    grid=(_E2 // _BE2,),
    in_specs=[
        pl.BlockSpec((_BE2, _D), lambda i: (i, 0)),
        pl.BlockSpec((_BE2, _D), lambda i: (i, 0)),
        pl.BlockSpec((_D, _D), lambda i: (0, 0)),
    ],
    out_specs=pl.BlockSpec((_BE2, _D), lambda i: (i, 0)),
    out_shape=jax.ShapeDtypeStruct((_E2, _D), jnp.float32),
)


def _node_update_body(cn_ref, nf_ref, p0_ref, p1_ref, wn1_ref, wna_ref,
                      out_ref):
    aggr = p0_ref[...] + p1_ref[...]
    v = (cn_ref[...]
         + jnp.dot(nf_ref[...], wn1_ref[...], preferred_element_type=jnp.float32)
         + jnp.dot(aggr, wna_ref[...], preferred_element_type=jnp.float32))
    out_ref[...] = jnp.maximum(v, 0.0)


_node_update = pl.pallas_call(
    _node_update_body,
    grid=(_N // _BN,),
    in_specs=[
        pl.BlockSpec((_BN, _D), lambda i: (i, 0)),
        pl.BlockSpec((_BN, _D), lambda i: (i, 0)),
        pl.BlockSpec((_BN, _DE), lambda i: (i, 0)),
        pl.BlockSpec((_BN, _DE), lambda i: (i, 0)),
        pl.BlockSpec((_D, _D), lambda i: (0, 0)),
        pl.BlockSpec((_DE, _D), lambda i: (0, 0)),
    ],
    out_specs=pl.BlockSpec((_BN, _D), lambda i: (i, 0)),
    out_shape=jax.ShapeDtypeStruct((_N, _D), jnp.float32),
)


def _head_body(nf_ref, w_ref, b_ref, out_ref):
    out_ref[...] = jnp.dot(nf_ref[...], w_ref[...],
                           preferred_element_type=jnp.float32) + b_ref[...]


def _make_head(k):
    return pl.pallas_call(
        _head_body,
        grid=(_N // _BN,),
        in_specs=[
            pl.BlockSpec((_BN, _D), lambda i: (i, 0)),
            pl.BlockSpec((_D, k), lambda i: (0, 0)),
            pl.BlockSpec((1, k), lambda i: (0, 0)),
        ],
        out_specs=pl.BlockSpec((_BN, k), lambda i: (i, 0)),
        out_shape=jax.ShapeDtypeStruct((_N, k), jnp.float32),
    )


_head1 = _make_head(1)
_head18 = _make_head(18)


def _edge_sc_body(as_hbm, ad_hbm, t_hbm, src_hbm, dst_hbm, ef_out, part_out,
                  sidx, didx, gs, gd, tb, sb, so, stage, accum,
                  semi, semg0, semg1, semt0, semt1, semw0, semw1,
                  semc0, semc1):
    cid = lax.axis_index("c")
    sid = lax.axis_index("s")
    wid = sid * _NC + cid
    semg = (semg0, semg1)
    semt = (semt0, semt1)
    semw = (semw0, semw1)
    semc = (semc0, semc1)

    hi1 = pltpu.async_copy(src_hbm.at[pl.ds(wid * _NCH, _NCH)], sidx, semi)
    hi2 = pltpu.async_copy(dst_hbm.at[pl.ds(wid * _NCH, _NCH)], didx, semi)

    def zrow(r, _):
        for j in range(4):
            stage[r, pl.ds(j * 16, 16)] = jnp.zeros((16,), jnp.float32)
        return 0

    lax.fori_loop(0, _ZR, zrow, 0)
    for k in range(_NPS // _ZR):
        pltpu.sync_copy(stage, accum.at[pl.ds(sid * _NPS + k * _ZR, _ZR)])

    @pl.when(sid == _NS - 1)
    def _():
        pltpu.sync_copy(stage.at[pl.ds(0, _TAIL)],
                        accum.at[pl.ds(_NPS * _NS, _TAIL)])

    hi1.wait()
    hi2.wait()
    plsc.subcore_barrier()

    def issue_in(i, b):
        pltpu.async_copy(as_hbm.at[sidx.at[i]], gs.at[b], semg[b])
        pltpu.async_copy(ad_hbm.at[didx.at[i]], gd.at[b], semg[b])
        pltpu.async_copy(t_hbm.at[pl.ds(wid * _E2PW + i * _CH2, _CH2)],
                         tb.at[b], semt[b])

    def wait_in(i, b):
        pltpu.make_async_copy(as_hbm.at[sidx.at[i]], gs.at[b], semg[b]).wait()
        pltpu.make_async_copy(ad_hbm.at[didx.at[i]], gd.at[b], semg[b]).wait()
        pltpu.make_async_copy(t_hbm.at[pl.ds(wid * _E2PW + i * _CH2, _CH2)],
                              tb.at[b], semt[b]).wait()

    def emit_out(i, b):
        pltpu.async_copy(so.at[b],
                         ef_out.at[pl.ds(wid * _E2PW + i * _CH2, _CH2)],
                         semw[b])
        pltpu.async_copy(sb.at[b], accum.at[didx.at[i]], semc[b], add=True)

    def wait_out(i, b):
        pltpu.make_async_copy(
            so.at[b], ef_out.at[pl.ds(wid * _E2PW + i * _CH2, _CH2)],
            semw[b]).wait()
        pltpu.make_async_copy(sb.at[b], accum.at[didx.at[i]], semc[b]).wait()

    def compute(b):
        def rowf(rr, _):
            for jj in range(8):
                e = 2 * rr + jj // 4
                sl = pl.ds((jj % 4) * 16, 16)
                v = jnp.maximum(
                    gs[b, e, sl] + gd[b, e, sl] + tb[b, rr,
                                                     pl.ds(jj * 16, 16)],
                    0.0)
                sb[b, e, sl] = v
                so[b, rr, pl.ds(jj * 16, 16)] = v
            return 0

        lax.fori_loop(0, _CH2, rowf, 0)

    def step(i, b, t):
        @pl.when(t > 0)
        def _():
            wait_out(i - 2, b)

        wait_in(i, b)
        compute(b)
        emit_out(i, b)

        @pl.when(i + 2 < _NCH)
        def _():
            issue_in(i + 2, b)

    issue_in(0, 0)
    issue_in(1, 1)

    def pair(t, _):
        step(2 * t, 0, t)
        step(2 * t + 1, 1, t)
        return 0

    lax.fori_loop(0, (_NCH - 1) // 2, pair, 0)
    last = _NCH - 1
    wait_out(last - 2, 0)
    wait_in(last, 0)
    compute(0)
    emit_out(last, 0)
    wait_out(last - 1, 1)
    wait_out(last, 0)
    plsc.subcore_barrier()

    for k in range(_NPS // _ZR):
        off = sid * _NPS + k * _ZR
        pltpu.sync_copy(accum.at[pl.ds(off, _ZR)], stage)
        pltpu.sync_copy(stage, part_out.at[cid, pl.ds(off, _ZR)])

    @pl.when(sid == _NS - 1)
    def _():
        off = _NPS * _NS
        pltpu.sync_copy(accum.at[pl.ds(off, _TAIL)], stage.at[pl.ds(0, _TAIL)])
        pltpu.sync_copy(stage.at[pl.ds(0, _TAIL)],
                        part_out.at[cid, pl.ds(off, _TAIL)])


@functools.cache
def _edge_sc():
    return pl.kernel(
        _edge_sc_body,
        mesh=plsc.VectorSubcoreMesh(core_axis_name="c", subcore_axis_name="s"),
        compiler_params=pltpu.CompilerParams(use_tc_tiling_on_sc=False),
        out_type=[
            jax.ShapeDtypeStruct((_E2, _D), jnp.float32),
            jax.ShapeDtypeStruct((_NC, _N, _DE), jnp.float32),
        ],
        scratch_types=[
            pltpu.VMEM((_NCH, _CH), jnp.int32),
            pltpu.VMEM((_NCH, _CH), jnp.int32),
            pltpu.VMEM((2, _CH, _DE), jnp.float32),
            pltpu.VMEM((2, _CH, _DE), jnp.float32),
            pltpu.VMEM((2, _CH2, _D), jnp.float32),
            pltpu.VMEM((2, _CH, _DE), jnp.float32),
            pltpu.VMEM((2, _CH2, _D), jnp.float32),
            pltpu.VMEM((_ZR, _DE), jnp.float32),
            pltpu.VMEM_SHARED((_N, _DE), jnp.float32),
        ] + [pltpu.SemaphoreType.DMA] * 9,
    )


def kernel(x, edge_attr, edge_index, node_types, W_ne, b_ne, W_ee, b_ee,
           W_e, b_e, W_n, b_n, W_tag, b_tag, W_nc, b_nc, W_cls, b_cls):
    del node_types
    src = edge_index[0].astype(jnp.int32).reshape(_NW * _NCH, _CH)
    dst = edge_index[1].astype(jnp.int32).reshape(_NW * _NCH, _CH)

    We_s0, We_s1 = W_e[0:128], W_e[128:256]
    We_d0, We_d1 = W_e[256:384], W_e[384:512]
    We_e0, We_e1 = W_e[512:576], W_e[576:640]
    Wn_0, Wn_1, Wn_a = W_n[0:128], W_n[128:256], W_n[256:320]

    def _bdiag(w):
        z = jnp.zeros_like(w)
        return jnp.concatenate(
            [jnp.concatenate([w, z], axis=1),
             jnp.concatenate([z, w], axis=1)], axis=0)

    def _btile(b):
        return jnp.concatenate([b, b]).reshape(1, -1)

    nf0, A0s, A0d, Cn = _encode_nodes(
        x, W_ne, b_ne.reshape(1, -1), We_s0, We_d0, Wn_0, b_n.reshape(1, -1))
    eat = edge_attr.T
    ef0, T0 = _encode_edges(
        eat[:, 0::2], eat[:, 1::2], W_ee, b_ee.reshape(1, -1),
        _bdiag(We_e0), _btile(b_e))

    nf, ef = nf0, ef0
    pred_tag = None
    We_e1d = _bdiag(We_e1)
    for layer in range(4):
        As, Ad = _proj(nf, A0s, A0d, We_s1, We_d1)
        T = _edge_t(T0, ef, We_e1d)
        ef, parts = _edge_sc()(As, Ad, T, src, dst)
        nf = _node_update(Cn, nf, parts[0], parts[1], Wn_1, Wn_a)
        if layer == 1:
            pred_tag = _head1(nf, W_tag, b_tag.reshape(1, -1))[:, 0]

    Wh = jnp.concatenate([W_nc, W_cls], axis=1)
    bh = jnp.concatenate([b_nc, b_cls]).reshape(1, -1)
    out = _head18(nf, Wh, bh)
    return out[:, 0], out[:, 1:], pred_tag


# speedup vs baseline: 1.3458x; 1.3458x over previous
import functools

import jax
import jax.numpy as jnp
from jax import lax
from jax.experimental import pallas as pl
from jax.experimental.pallas import tpu as pltpu
from jax.experimental.pallas import tpu_sc as plsc

_N = 10000
_E = 320000
_E2 = _E // 2
_D = 128
_DE = 64

_BN = 2000
_BE2 = 1280

_NC = 2
_NS = 16
_NW = _NC * _NS
_EPW = _E // _NW
_CH = 80
_CH2 = _CH // 2
_NCH = _EPW // _CH
_E2PW = _E2 // _NW
_NPS = 624
_ZR = 104
_TAIL = _N - _NPS * _NS

_CA = 63
_CB = _NCH - _CA
_GA = _NW * _CA
_E2A = _GA * _CH2
_E2B = _E2 - _E2A


def _encode_nodes_body(x_ref, wne_ref, bne_ref, ws0_ref, wd0_ref, wn0_ref,
                       bn_ref, nf0_ref, a0s_ref, a0d_ref, cn_ref):
    nf0 = jnp.dot(x_ref[...], wne_ref[...],
                  preferred_element_type=jnp.float32) + bne_ref[...]
    nf0_ref[...] = nf0
    a0s_ref[...] = jnp.dot(nf0, ws0_ref[...], preferred_element_type=jnp.float32)
    a0d_ref[...] = jnp.dot(nf0, wd0_ref[...], preferred_element_type=jnp.float32)
    cn_ref[...] = jnp.dot(nf0, wn0_ref[...],
                          preferred_element_type=jnp.float32) + bn_ref[...]


_encode_nodes = pl.pallas_call(
    _encode_nodes_body,
    grid=(_N // _BN,),
    in_specs=[
        pl.BlockSpec((_BN, _D), lambda i: (i, 0)),
        pl.BlockSpec((_D, _D), lambda i: (0, 0)),
        pl.BlockSpec((1, _D), lambda i: (0, 0)),
        pl.BlockSpec((_D, _DE), lambda i: (0, 0)),
        pl.BlockSpec((_D, _DE), lambda i: (0, 0)),
        pl.BlockSpec((_D, _D), lambda i: (0, 0)),
        pl.BlockSpec((1, _D), lambda i: (0, 0)),
    ],
    out_specs=[
        pl.BlockSpec((_BN, _D), lambda i: (i, 0)),
        pl.BlockSpec((_BN, _DE), lambda i: (i, 0)),
        pl.BlockSpec((_BN, _DE), lambda i: (i, 0)),
        pl.BlockSpec((_BN, _D), lambda i: (i, 0)),
    ],
    out_shape=[
        jax.ShapeDtypeStruct((_N, _D), jnp.float32),
        jax.ShapeDtypeStruct((_N, _DE), jnp.float32),
        jax.ShapeDtypeStruct((_N, _DE), jnp.float32),
        jax.ShapeDtypeStruct((_N, _D), jnp.float32),
    ],
)


def _encode_edges_body(ea_ref, wee_ref, bee_ref, we0_ref, be_ref,
                       ef0_ref, t0_ref):
    ef0 = jnp.dot(ea_ref[...], wee_ref[...],
                  preferred_element_type=jnp.float32) + bee_ref[...]
    ef0_ref[...] = ef0
    t0_ref[...] = jnp.dot(ef0, we0_ref[...],
                          preferred_element_type=jnp.float32) + be_ref[...]


def _make_encode_edges(nblk, off):
    return pl.pallas_call(
        _encode_edges_body,
        grid=(nblk,),
        in_specs=[
            pl.BlockSpec((_BE2, 32), lambda i: (i + off, 0)),
            pl.BlockSpec((32, _D), lambda i: (0, 0)),
            pl.BlockSpec((1, _D), lambda i: (0, 0)),
            pl.BlockSpec((_D, _D), lambda i: (0, 0)),
            pl.BlockSpec((1, _D), lambda i: (0, 0)),
        ],
        out_specs=[
            pl.BlockSpec((_BE2, _D), lambda i: (i, 0)),
            pl.BlockSpec((_BE2, _D), lambda i: (i, 0)),
        ],
        out_shape=[
            jax.ShapeDtypeStruct((nblk * _BE2, _D), jnp.float32),
            jax.ShapeDtypeStruct((nblk * _BE2, _D), jnp.float32),
        ],
    )


_encode_edges_a = _make_encode_edges(_E2A // _BE2, 0)
_encode_edges_b = _make_encode_edges(_E2B // _BE2, _E2A // _BE2)


def _proj_body(nf_ref, a0s_ref, a0d_ref, ws1_ref, wd1_ref, as_ref, ad_ref):
    nf = nf_ref[...]
    as_ref[...] = a0s_ref[...] + jnp.dot(nf, ws1_ref[...],
                                         preferred_element_type=jnp.float32)
    ad_ref[...] = a0d_ref[...] + jnp.dot(nf, wd1_ref[...],
                                         preferred_element_type=jnp.float32)


_proj = pl.pallas_call(
    _proj_body,
    grid=(_N // _BN,),
    in_specs=[
        pl.BlockSpec((_BN, _D), lambda i: (i, 0)),
        pl.BlockSpec((_BN, _DE), lambda i: (i, 0)),
        pl.BlockSpec((_BN, _DE), lambda i: (i, 0)),
        pl.BlockSpec((_D, _DE), lambda i: (0, 0)),
        pl.BlockSpec((_D, _DE), lambda i: (0, 0)),
    ],
    out_specs=[
        pl.BlockSpec((_BN, _DE), lambda i: (i, 0)),
        pl.BlockSpec((_BN, _DE), lambda i: (i, 0)),
    ],
    out_shape=[
        jax.ShapeDtypeStruct((_N, _DE), jnp.float32),
        jax.ShapeDtypeStruct((_N, _DE), jnp.float32),
    ],
)


def _edge_t_body(t0_ref, ef_ref, we1_ref, t_ref):
    t_ref[...] = t0_ref[...] + jnp.dot(ef_ref[...], we1_ref[...],
                                       preferred_element_type=jnp.float32)


def _make_edge_t(nblk):
    return pl.pallas_call(
        _edge_t_body,
        grid=(nblk,),
        in_specs=[
            pl.BlockSpec((_BE2, _D), lambda i: (i, 0)),
            pl.BlockSpec((_BE2, _D), lambda i: (i, 0)),
            pl.BlockSpec((_D, _D), lambda i: (0, 0)),
        ],
        out_specs=pl.BlockSpec((_BE2, _D), lambda i: (i, 0)),
        out_shape=jax.ShapeDtypeStruct((nblk * _BE2, _D), jnp.float32),
    )


_edge_t_a = _make_edge_t(_E2A // _BE2)
_edge_t_b = _make_edge_t(_E2B // _BE2)


def _node_update_body(cn_ref, nf_ref, p0_ref, p1_ref, wn1_ref, wna_ref,
                      out_ref):
    aggr = p0_ref[...] + p1_ref[...]
    v = (cn_ref[...]
         + jnp.dot(nf_ref[...], wn1_ref[...], preferred_element_type=jnp.float32)
         + jnp.dot(aggr, wna_ref[...], preferred_element_type=jnp.float32))
    out_ref[...] = jnp.maximum(v, 0.0)


_node_update = pl.pallas_call(
    _node_update_body,
    grid=(_N // _BN,),
    in_specs=[
        pl.BlockSpec((_BN, _D), lambda i: (i, 0)),
        pl.BlockSpec((_BN, _D), lambda i: (i, 0)),
        pl.BlockSpec((_BN, _DE), lambda i: (i, 0)),
        pl.BlockSpec((_BN, _DE), lambda i: (i, 0)),
        pl.BlockSpec((_D, _D), lambda i: (0, 0)),
        pl.BlockSpec((_DE, _D), lambda i: (0, 0)),
    ],
    out_specs=pl.BlockSpec((_BN, _D), lambda i: (i, 0)),
    out_shape=jax.ShapeDtypeStruct((_N, _D), jnp.float32),
)


def _head_body(nf_ref, w_ref, b_ref, out_ref):
    out_ref[...] = jnp.dot(nf_ref[...], w_ref[...],
                           preferred_element_type=jnp.float32) + b_ref[...]


def _make_head(k):
    return pl.pallas_call(
        _head_body,
        grid=(_N // _BN,),
        in_specs=[
            pl.BlockSpec((_BN, _D), lambda i: (i, 0)),
            pl.BlockSpec((_D, k), lambda i: (0, 0)),
            pl.BlockSpec((1, k), lambda i: (0, 0)),
        ],
        out_specs=pl.BlockSpec((_BN, k), lambda i: (i, 0)),
        out_shape=jax.ShapeDtypeStruct((_N, k), jnp.float32),
    )


_head1 = _make_head(1)
_head18 = _make_head(18)


def _make_sc_body(cpw, goff, from_partial):
    def body(as_hbm, ad_hbm, t_hbm, src_hbm, dst_hbm, *rest):
        if from_partial:
            (pa_hbm, ef_out, part_out, sidx, didx, gs, gd, tb, sb, so,
             stage, accum, semi, semg0, semg1, semt0, semt1, semw0, semw1,
             semc0, semc1) = rest
        else:
            (ef_out, part_out, sidx, didx, gs, gd, tb, sb, so,
             stage, accum, semi, semg0, semg1, semt0, semt1, semw0, semw1,
             semc0, semc1) = rest
        cid = lax.axis_index("c")
        sid = lax.axis_index("s")
        wid = sid * _NC + cid
        semg = (semg0, semg1)
        semt = (semt0, semt1)
        semw = (semw0, semw1)
        semc = (semc0, semc1)

        hi1 = pltpu.async_copy(
            src_hbm.at[pl.ds(goff + wid * cpw, cpw)], sidx, semi)
        hi2 = pltpu.async_copy(
            dst_hbm.at[pl.ds(goff + wid * cpw, cpw)], didx, semi)

        if from_partial:
            for k in range(_NPS // _ZR):
                off = sid * _NPS + k * _ZR
                pltpu.sync_copy(pa_hbm.at[cid, pl.ds(off, _ZR)], stage)
                pltpu.sync_copy(stage, accum.at[pl.ds(off, _ZR)])

            @pl.when(sid == _NS - 1)
            def _():
                off = _NPS * _NS
                pltpu.sync_copy(pa_hbm.at[cid, pl.ds(off, _TAIL)],
                                stage.at[pl.ds(0, _TAIL)])
                pltpu.sync_copy(stage.at[pl.ds(0, _TAIL)],
                                accum.at[pl.ds(off, _TAIL)])
        else:
            def zrow(r, _):
                for j in range(4):
                    stage[r, pl.ds(j * 16, 16)] = jnp.zeros((16,),
                                                            jnp.float32)
                return 0

            lax.fori_loop(0, _ZR, zrow, 0)
            for k in range(_NPS // _ZR):
                pltpu.sync_copy(stage,
                                accum.at[pl.ds(sid * _NPS + k * _ZR, _ZR)])

            @pl.when(sid == _NS - 1)
            def _():
                pltpu.sync_copy(stage.at[pl.ds(0, _TAIL)],
                                accum.at[pl.ds(_NPS * _NS, _TAIL)])

        hi1.wait()
        hi2.wait()
        plsc.subcore_barrier()

        def t_rows(i):
            return pl.ds((wid * cpw + i) * _CH2, _CH2)

        def issue_in(i, b):
            pltpu.async_copy(as_hbm.at[sidx.at[i]], gs.at[b], semg[b])
            pltpu.async_copy(ad_hbm.at[didx.at[i]], gd.at[b], semg[b])
            pltpu.async_copy(t_hbm.at[t_rows(i)], tb.at[b], semt[b])

        def wait_in(i, b):
            pltpu.make_async_copy(as_hbm.at[sidx.at[i]], gs.at[b],
                                  semg[b]).wait()
            pltpu.make_async_copy(ad_hbm.at[didx.at[i]], gd.at[b],
                                  semg[b]).wait()
            pltpu.make_async_copy(t_hbm.at[t_rows(i)], tb.at[b],
                                  semt[b]).wait()

        def emit_out(i, b):
            pltpu.async_copy(so.at[b], ef_out.at[t_rows(i)], semw[b])
            pltpu.async_copy(sb.at[b], accum.at[didx.at[i]], semc[b],
                             add=True)

        def wait_out(i, b):
            pltpu.make_async_copy(so.at[b], ef_out.at[t_rows(i)],
                                  semw[b]).wait()
            pltpu.make_async_copy(sb.at[b], accum.at[didx.at[i]],
                                  semc[b]).wait()

        def compute(b):
            def rowf(rr, _):
                for jj in range(8):
                    e = 2 * rr + jj // 4
                    sl = pl.ds((jj % 4) * 16, 16)
                    v = jnp.maximum(
                        gs[b, e, sl] + gd[b, e, sl]
                        + tb[b, rr, pl.ds(jj * 16, 16)], 0.0)
                    sb[b, e, sl] = v
                    so[b, rr, pl.ds(jj * 16, 16)] = v
                return 0

            lax.fori_loop(0, _CH2, rowf, 0)

        def step(i, b, t):
            @pl.when(t > 0)
            def _():
                wait_out(i - 2, b)

            wait_in(i, b)
            compute(b)
            emit_out(i, b)

            @pl.when(i + 2 < cpw)
            def _():
                issue_in(i + 2, b)

        issue_in(0, 0)
        issue_in(1, 1)

        def pair(t, _):
            step(2 * t, 0, t)
            step(2 * t + 1, 1, t)
            return 0

        lax.fori_loop(0, cpw // 2, pair, 0)
        if cpw % 2:
            last = cpw - 1
            wait_out(last - 2, 0)
            wait_in(last, 0)
            compute(0)
            emit_out(last, 0)
            wait_out(last - 1, 1)
            wait_out(last, 0)
        else:
            wait_out(cpw - 2, 0)
            wait_out(cpw - 1, 1)
        plsc.subcore_barrier()

        for k in range(_NPS // _ZR):
            off = sid * _NPS + k * _ZR
            pltpu.sync_copy(accum.at[pl.ds(off, _ZR)], stage)
            pltpu.sync_copy(stage, part_out.at[cid, pl.ds(off, _ZR)])

        @pl.when(sid == _NS - 1)
        def _():
            off = _NPS * _NS
            pltpu.sync_copy(accum.at[pl.ds(off, _TAIL)],
                            stage.at[pl.ds(0, _TAIL)])
            pltpu.sync_copy(stage.at[pl.ds(0, _TAIL)],
                            part_out.at[cid, pl.ds(off, _TAIL)])

    return body


@functools.cache
def _edge_sc(half):
    cpw = _CA if half == 0 else _CB
    goff = 0 if half == 0 else _GA
    nrows = _E2A if half == 0 else _E2B
    return pl.kernel(
        _make_sc_body(cpw, goff, from_partial=(half == 1)),
        mesh=plsc.VectorSubcoreMesh(core_axis_name="c", subcore_axis_name="s"),
        compiler_params=pltpu.CompilerParams(use_tc_tiling_on_sc=False),
        out_type=[
            jax.ShapeDtypeStruct((nrows, _D), jnp.float32),
            jax.ShapeDtypeStruct((_NC, _N, _DE), jnp.float32),
        ],
        scratch_types=[
            pltpu.VMEM((cpw, _CH), jnp.int32),
            pltpu.VMEM((cpw, _CH), jnp.int32),
            pltpu.VMEM((2, _CH, _DE), jnp.float32),
            pltpu.VMEM((2, _CH, _DE), jnp.float32),
            pltpu.VMEM((2, _CH2, _D), jnp.float32),
            pltpu.VMEM((2, _CH, _DE), jnp.float32),
            pltpu.VMEM((2, _CH2, _D), jnp.float32),
            pltpu.VMEM((_ZR, _DE), jnp.float32),
            pltpu.VMEM_SHARED((_N, _DE), jnp.float32),
        ] + [pltpu.SemaphoreType.DMA] * 9,
    )


def kernel(x, edge_attr, edge_index, node_types, W_ne, b_ne, W_ee, b_ee,
           W_e, b_e, W_n, b_n, W_tag, b_tag, W_nc, b_nc, W_cls, b_cls):
    del node_types
    src = edge_index[0].astype(jnp.int32).reshape(_NW * _NCH, _CH)
    dst = edge_index[1].astype(jnp.int32).reshape(_NW * _NCH, _CH)

    We_s0, We_s1 = W_e[0:128], W_e[128:256]
    We_d0, We_d1 = W_e[256:384], W_e[384:512]
    We_e0, We_e1 = W_e[512:576], W_e[576:640]
    Wn_0, Wn_1, Wn_a = W_n[0:128], W_n[128:256], W_n[256:320]

    def _bdiag(w):
        z = jnp.zeros_like(w)
        return jnp.concatenate(
            [jnp.concatenate([w, z], axis=1),
             jnp.concatenate([z, w], axis=1)], axis=0)

    def _btile(b):
        return jnp.concatenate([b, b]).reshape(1, -1)

    nf0, A0s, A0d, Cn = _encode_nodes(
        x, W_ne, b_ne.reshape(1, -1), We_s0, We_d0, Wn_0, b_n.reshape(1, -1))
    ea2 = edge_attr.reshape(_E2, 32)
    efa, T0a = _encode_edges_a(ea2, _bdiag(W_ee), _btile(b_ee),
                               _bdiag(We_e0), _btile(b_e))
    efb, T0b = _encode_edges_b(ea2, _bdiag(W_ee), _btile(b_ee),
                               _bdiag(We_e0), _btile(b_e))

    nf = nf0
    pred_tag = None
    We_e1d = _bdiag(We_e1)
    for layer in range(4):
        As, Ad = _proj(nf, A0s, A0d, We_s1, We_d1)
        Ta = _edge_t_a(T0a, efa, We_e1d)
        Tb = _edge_t_b(T0b, efb, We_e1d)
        efa, pa = _edge_sc(0)(As, Ad, Ta, src, dst)
        efb, parts = _edge_sc(1)(As, Ad, Tb, src, dst, pa)
        nf = _node_update(Cn, nf, parts[0], parts[1], Wn_1, Wn_a)
        if layer == 1:
            pred_tag = _head1(nf, W_tag, b_tag.reshape(1, -1))[:, 0]

    Wh = jnp.concatenate([W_nc, W_cls], axis=1)
    bh = jnp.concatenate([b_nc, b_cls]).reshape(1, -1)
    out = _head18(nf, Wh, bh)
    return out[:, 0], out[:, 1:], pred_tag


# speedup vs baseline: 1.4054x; 1.0443x over previous
import functools

import jax
import jax.numpy as jnp
from jax import lax
from jax.experimental import pallas as pl
from jax.experimental.pallas import tpu as pltpu
from jax.experimental.pallas import tpu_sc as plsc

_N = 10000
_E = 320000
_E2 = _E // 2
_D = 128
_DE = 64

_BN = 2000
_BE2 = 3200

_NC = 2
_NS = 16
_NW = _NC * _NS
_EPW = _E // _NW
_CH = 80
_CH2 = _CH // 2
_NCH = _EPW // _CH
_E2PW = _E2 // _NW
_NPS = 624
_ZR = 104
_TAIL = _N - _NPS * _NS


def _encode_nodes_body(x_ref, wne_ref, bne_ref, ws0_ref, wd0_ref, wn0_ref,
                       bn_ref, nf0_ref, a0s_ref, a0d_ref, cn_ref):
    nf0 = jnp.dot(x_ref[...], wne_ref[...],
                  preferred_element_type=jnp.float32) + bne_ref[...]
    nf0_ref[...] = nf0
    a0s_ref[...] = jnp.dot(nf0, ws0_ref[...], preferred_element_type=jnp.float32)
    a0d_ref[...] = jnp.dot(nf0, wd0_ref[...], preferred_element_type=jnp.float32)
    cn_ref[...] = jnp.dot(nf0, wn0_ref[...],
                          preferred_element_type=jnp.float32) + bn_ref[...]


_encode_nodes = pl.pallas_call(
    _encode_nodes_body,
    grid=(_N // _BN,),
    in_specs=[
        pl.BlockSpec((_BN, _D), lambda i: (i, 0)),
        pl.BlockSpec((_D, _D), lambda i: (0, 0)),
        pl.BlockSpec((1, _D), lambda i: (0, 0)),
        pl.BlockSpec((_D, _DE), lambda i: (0, 0)),
        pl.BlockSpec((_D, _DE), lambda i: (0, 0)),
        pl.BlockSpec((_D, _D), lambda i: (0, 0)),
        pl.BlockSpec((1, _D), lambda i: (0, 0)),
    ],
    out_specs=[
        pl.BlockSpec((_BN, _D), lambda i: (i, 0)),
        pl.BlockSpec((_BN, _DE), lambda i: (i, 0)),
        pl.BlockSpec((_BN, _DE), lambda i: (i, 0)),
        pl.BlockSpec((_BN, _D), lambda i: (i, 0)),
    ],
    out_shape=[
        jax.ShapeDtypeStruct((_N, _D), jnp.float32),
        jax.ShapeDtypeStruct((_N, _DE), jnp.float32),
        jax.ShapeDtypeStruct((_N, _DE), jnp.float32),
        jax.ShapeDtypeStruct((_N, _D), jnp.float32),
    ],
)


def _encode_edges_body(ea_ref, wee_ref, bee_ref, we0_ref, be_ref,
                       ef0_ref, t0_ref):
    ef0 = jnp.dot(ea_ref[...], wee_ref[...],
                  preferred_element_type=jnp.float32) + bee_ref[...]
    ef0_ref[...] = ef0
    t0_ref[...] = jnp.dot(ef0, we0_ref[...],
                          preferred_element_type=jnp.float32) + be_ref[...]


_encode_edges = pl.pallas_call(
    _encode_edges_body,
    grid=(_E2 // _BE2,),
    in_specs=[
        pl.BlockSpec((_BE2, 32), lambda i: (i, 0)),
        pl.BlockSpec((32, _D), lambda i: (0, 0)),
        pl.BlockSpec((1, _D), lambda i: (0, 0)),
        pl.BlockSpec((_D, _D), lambda i: (0, 0)),
        pl.BlockSpec((1, _D), lambda i: (0, 0)),
    ],
    out_specs=[
        pl.BlockSpec((_BE2, _D), lambda i: (i, 0)),
        pl.BlockSpec((_BE2, _D), lambda i: (i, 0)),
    ],
    out_shape=[
        jax.ShapeDtypeStruct((_E2, _D), jnp.float32),
        jax.ShapeDtypeStruct((_E2, _D), jnp.float32),
    ],
)


def _proj_body(nf_ref, a0s_ref, a0d_ref, ws1_ref, wd1_ref, as_ref, ad_ref):
    nf = nf_ref[...]
    as_ref[...] = a0s_ref[...] + jnp.dot(nf, ws1_ref[...],
                                         preferred_element_type=jnp.float32)
    ad_ref[...] = a0d_ref[...] + jnp.dot(nf, wd1_ref[...],
                                         preferred_element_type=jnp.float32)


_proj = pl.pallas_call(
    _proj_body,
    grid=(_N // _BN,),
    in_specs=[
        pl.BlockSpec((_BN, _D), lambda i: (i, 0)),
        pl.BlockSpec((_BN, _DE), lambda i: (i, 0)),
        pl.BlockSpec((_BN, _DE), lambda i: (i, 0)),
        pl.BlockSpec((_D, _DE), lambda i: (0, 0)),
        pl.BlockSpec((_D, _DE), lambda i: (0, 0)),
    ],
    out_specs=[
        pl.BlockSpec((_BN, _DE), lambda i: (i, 0)),
        pl.BlockSpec((_BN, _DE), lambda i: (i, 0)),
    ],
    out_shape=[
        jax.ShapeDtypeStruct((_N, _DE), jnp.float32),
        jax.ShapeDtypeStruct((_N, _DE), jnp.float32),
    ],
)


def _edge_t_body(t0_ref, ef_ref, we1_ref, t_ref):
    t_ref[...] = t0_ref[...] + jnp.dot(ef_ref[...], we1_ref[...],
                                       preferred_element_type=jnp.float32)


_edge_t = pl.pallas_call(
    _edge_t_body,
    grid=(_E2 // _BE2,),
    in_specs=[
        pl.BlockSpec((_BE2, _D), lambda i: (i, 0)),
        pl.BlockSpec((_BE2, _D), lambda i: (i, 0)),
        pl.BlockSpec((_D, _D), lambda i: (0, 0)),
    ],
    out_specs=pl.BlockSpec((_BE2, _D), lambda i: (i, 0)),
    out_shape=jax.ShapeDtypeStruct((_E2, _D), jnp.float32),
)


def _node_update_body(cn_ref, nf_ref, p0_ref, p1_ref, wn1_ref, wna_ref,
                      out_ref):
    aggr = p0_ref[...] + p1_ref[...]
    v = (cn_ref[...]
         + jnp.dot(nf_ref[...], wn1_ref[...], preferred_element_type=jnp.float32)
         + jnp.dot(aggr, wna_ref[...], preferred_element_type=jnp.float32))
    out_ref[...] = jnp.maximum(v, 0.0)


_node_update = pl.pallas_call(
    _node_update_body,
    grid=(_N // _BN,),
    in_specs=[
        pl.BlockSpec((_BN, _D), lambda i: (i, 0)),
        pl.BlockSpec((_BN, _D), lambda i: (i, 0)),
        pl.BlockSpec((_BN, _DE), lambda i: (i, 0)),
        pl.BlockSpec((_BN, _DE), lambda i: (i, 0)),
        pl.BlockSpec((_D, _D), lambda i: (0, 0)),
        pl.BlockSpec((_DE, _D), lambda i: (0, 0)),
    ],
    out_specs=pl.BlockSpec((_BN, _D), lambda i: (i, 0)),
    out_shape=jax.ShapeDtypeStruct((_N, _D), jnp.float32),
)


def _head_body(nf_ref, w_ref, b_ref, out_ref):
    out_ref[...] = jnp.dot(nf_ref[...], w_ref[...],
                           preferred_element_type=jnp.float32) + b_ref[...]


def _make_head(k):
    return pl.pallas_call(
        _head_body,
        grid=(_N // _BN,),
        in_specs=[
            pl.BlockSpec((_BN, _D), lambda i: (i, 0)),
            pl.BlockSpec((_D, k), lambda i: (0, 0)),
            pl.BlockSpec((1, k), lambda i: (0, 0)),
        ],
        out_specs=pl.BlockSpec((_BN, k), lambda i: (i, 0)),
        out_shape=jax.ShapeDtypeStruct((_N, k), jnp.float32),
    )


_head1 = _make_head(1)
_head18 = _make_head(18)


def _edge_sc_body(as_hbm, ad_hbm, t_hbm, src_hbm, dst_hbm, ef_out, part_out,
                  sidx, didx, gs, gd, tb, sb, so, stage, accum,
                  semi, semg0, semg1, semt0, semt1, semw0, semw1,
                  semc0, semc1):
    cid = lax.axis_index("c")
    sid = lax.axis_index("s")
    wid = sid * _NC + cid
    semg = (semg0, semg1)
    semt = (semt0, semt1)
    semw = (semw0, semw1)
    semc = (semc0, semc1)

    hi1 = pltpu.async_copy(src_hbm.at[pl.ds(wid * _NCH, _NCH)], sidx, semi)
    hi2 = pltpu.async_copy(dst_hbm.at[pl.ds(wid * _NCH, _NCH)], didx, semi)

    def zrow(r, _):
        for j in range(4):
            stage[r, pl.ds(j * 16, 16)] = jnp.zeros((16,), jnp.float32)
        return 0

    lax.fori_loop(0, _ZR, zrow, 0)
    for k in range(_NPS // _ZR):
        pltpu.sync_copy(stage, accum.at[pl.ds(sid * _NPS + k * _ZR, _ZR)])

    @pl.when(sid == _NS - 1)
    def _():
        pltpu.sync_copy(stage.at[pl.ds(0, _TAIL)],
                        accum.at[pl.ds(_NPS * _NS, _TAIL)])

    hi1.wait()
    hi2.wait()
    plsc.subcore_barrier()

    def issue_in(i, b):
        pltpu.async_copy(as_hbm.at[sidx.at[i]], gs.at[b], semg[b])
        pltpu.async_copy(ad_hbm.at[didx.at[i]], gd.at[b], semg[b])
        pltpu.async_copy(t_hbm.at[pl.ds(wid * _E2PW + i * _CH2, _CH2)],
                         tb.at[b], semt[b])

    def wait_in(i, b):
        pltpu.make_async_copy(as_hbm.at[sidx.at[i]], gs.at[b], semg[b]).wait()
        pltpu.make_async_copy(ad_hbm.at[didx.at[i]], gd.at[b], semg[b]).wait()
        pltpu.make_async_copy(t_hbm.at[pl.ds(wid * _E2PW + i * _CH2, _CH2)],
                              tb.at[b], semt[b]).wait()

    def emit_out(i, b):
        pltpu.async_copy(so.at[b],
                         ef_out.at[pl.ds(wid * _E2PW + i * _CH2, _CH2)],
                         semw[b])
        pltpu.async_copy(sb.at[b], accum.at[didx.at[i]], semc[b], add=True)

    def wait_out(i, b):
        pltpu.make_async_copy(
            so.at[b], ef_out.at[pl.ds(wid * _E2PW + i * _CH2, _CH2)],
            semw[b]).wait()
        pltpu.make_async_copy(sb.at[b], accum.at[didx.at[i]], semc[b]).wait()

    def compute(b):
        def rowf(rr, _):
            for jj in range(8):
                e = 2 * rr + jj // 4
                sl = pl.ds((jj % 4) * 16, 16)
                v = jnp.maximum(
                    gs[b, e, sl] + gd[b, e, sl] + tb[b, rr,
                                                     pl.ds(jj * 16, 16)],
                    0.0)
                sb[b, e, sl] = v
                so[b, rr, pl.ds(jj * 16, 16)] = v
            return 0

        lax.fori_loop(0, _CH2, rowf, 0)

    def step(i, b, t):
        @pl.when(t > 0)
        def _():
            wait_out(i - 2, b)

        wait_in(i, b)
        compute(b)
        emit_out(i, b)

        @pl.when(i + 2 < _NCH)
        def _():
            issue_in(i + 2, b)

    issue_in(0, 0)
    issue_in(1, 1)

    def pair(t, _):
        step(2 * t, 0, t)
        step(2 * t + 1, 1, t)
        return 0

    lax.fori_loop(0, (_NCH - 1) // 2, pair, 0)
    last = _NCH - 1
    wait_out(last - 2, 0)
    wait_in(last, 0)
    compute(0)
    emit_out(last, 0)
    wait_out(last - 1, 1)
    wait_out(last, 0)
    plsc.subcore_barrier()

    for k in range(_NPS // _ZR):
        off = sid * _NPS + k * _ZR
        pltpu.sync_copy(accum.at[pl.ds(off, _ZR)], stage)
        pltpu.sync_copy(stage, part_out.at[cid, pl.ds(off, _ZR)])

    @pl.when(sid == _NS - 1)
    def _():
        off = _NPS * _NS
        pltpu.sync_copy(accum.at[pl.ds(off, _TAIL)], stage.at[pl.ds(0, _TAIL)])
        pltpu.sync_copy(stage.at[pl.ds(0, _TAIL)],
                        part_out.at[cid, pl.ds(off, _TAIL)])


@functools.cache
def _edge_sc():
    return pl.kernel(
        _edge_sc_body,
        mesh=plsc.VectorSubcoreMesh(core_axis_name="c", subcore_axis_name="s"),
        compiler_params=pltpu.CompilerParams(use_tc_tiling_on_sc=False),
        out_type=[
            jax.ShapeDtypeStruct((_E2, _D), jnp.float32),
            jax.ShapeDtypeStruct((_NC, _N, _DE), jnp.float32),
        ],
        scratch_types=[
            pltpu.VMEM((_NCH, _CH), jnp.int32),
            pltpu.VMEM((_NCH, _CH), jnp.int32),
            pltpu.VMEM((2, _CH, _DE), jnp.float32),
            pltpu.VMEM((2, _CH, _DE), jnp.float32),
            pltpu.VMEM((2, _CH2, _D), jnp.float32),
            pltpu.VMEM((2, _CH, _DE), jnp.float32),
            pltpu.VMEM((2, _CH2, _D), jnp.float32),
            pltpu.VMEM((_ZR, _DE), jnp.float32),
            pltpu.VMEM_SHARED((_N, _DE), jnp.float32),
        ] + [pltpu.SemaphoreType.DMA] * 9,
    )


def kernel(x, edge_attr, edge_index, node_types, W_ne, b_ne, W_ee, b_ee,
           W_e, b_e, W_n, b_n, W_tag, b_tag, W_nc, b_nc, W_cls, b_cls):
    del node_types
    src = edge_index[0].astype(jnp.int32).reshape(_NW * _NCH, _CH)
    dst = edge_index[1].astype(jnp.int32).reshape(_NW * _NCH, _CH)

    We_s0, We_s1 = W_e[0:128], W_e[128:256]
    We_d0, We_d1 = W_e[256:384], W_e[384:512]
    We_e0, We_e1 = W_e[512:576], W_e[576:640]
    Wn_0, Wn_1, Wn_a = W_n[0:128], W_n[128:256], W_n[256:320]

    def _bdiag(w):
        z = jnp.zeros_like(w)
        return jnp.concatenate(
            [jnp.concatenate([w, z], axis=1),
             jnp.concatenate([z, w], axis=1)], axis=0)

    def _btile(b):
        return jnp.concatenate([b, b]).reshape(1, -1)

    nf0, A0s, A0d, Cn = _encode_nodes(
        x, W_ne, b_ne.reshape(1, -1), We_s0, We_d0, Wn_0, b_n.reshape(1, -1))
    ef0, T0 = _encode_edges(
        edge_attr.reshape(_E2, 32), _bdiag(W_ee), _btile(b_ee),
        _bdiag(We_e0), _btile(b_e))

    nf, ef = nf0, ef0
    pred_tag = None
    We_e1d = _bdiag(We_e1)
    for layer in range(4):
        As, Ad = _proj(nf, A0s, A0d, We_s1, We_d1)
        T = _edge_t(T0, ef, We_e1d)
        ef, parts = _edge_sc()(As, Ad, T, src, dst)
        nf = _node_update(Cn, nf, parts[0], parts[1], Wn_1, Wn_a)
        if layer == 1:
            pred_tag = _head1(nf, W_tag, b_tag.reshape(1, -1))[:, 0]

    Wh = jnp.concatenate([W_nc, W_cls], axis=1)
    bh = jnp.concatenate([b_nc, b_cls]).reshape(1, -1)
    out = _head18(nf, Wh, bh)
    return out[:, 0], out[:, 1:], pred_tag


# speedup vs baseline: 1.4206x; 1.0108x over previous
import functools

import jax
import jax.numpy as jnp
from jax import lax
from jax.experimental import pallas as pl
from jax.experimental.pallas import tpu as pltpu
from jax.experimental.pallas import tpu_sc as plsc

_N = 10000
_E = 320000
_E2 = _E // 2
_D = 128
_DE = 64

_BN = 2000
_BE2 = 4000

_NC = 2
_NS = 16
_NW = _NC * _NS
_EPW = _E // _NW
_CH = 80
_CH2 = _CH // 2
_NCH = _EPW // _CH
_E2PW = _E2 // _NW
_NPS = 624
_ZR = 104
_TAIL = _N - _NPS * _NS


def _encode_nodes_body(x_ref, wne_ref, bne_ref, ws0_ref, wd0_ref, wn0_ref,
                       bn_ref, nf0_ref, a0s_ref, a0d_ref, cn_ref):
    nf0 = jnp.dot(x_ref[...], wne_ref[...],
                  preferred_element_type=jnp.float32) + bne_ref[...]
    nf0_ref[...] = nf0
    a0s_ref[...] = jnp.dot(nf0, ws0_ref[...], preferred_element_type=jnp.float32)
    a0d_ref[...] = jnp.dot(nf0, wd0_ref[...], preferred_element_type=jnp.float32)
    cn_ref[...] = jnp.dot(nf0, wn0_ref[...],
                          preferred_element_type=jnp.float32) + bn_ref[...]


_encode_nodes = pl.pallas_call(
    _encode_nodes_body,
    grid=(_N // _BN,),
    in_specs=[
        pl.BlockSpec((_BN, _D), lambda i: (i, 0)),
        pl.BlockSpec((_D, _D), lambda i: (0, 0)),
        pl.BlockSpec((1, _D), lambda i: (0, 0)),
        pl.BlockSpec((_D, _DE), lambda i: (0, 0)),
        pl.BlockSpec((_D, _DE), lambda i: (0, 0)),
        pl.BlockSpec((_D, _D), lambda i: (0, 0)),
        pl.BlockSpec((1, _D), lambda i: (0, 0)),
    ],
    out_specs=[
        pl.BlockSpec((_BN, _D), lambda i: (i, 0)),
        pl.BlockSpec((_BN, _DE), lambda i: (i, 0)),
        pl.BlockSpec((_BN, _DE), lambda i: (i, 0)),
        pl.BlockSpec((_BN, _D), lambda i: (i, 0)),
    ],
    out_shape=[
        jax.ShapeDtypeStruct((_N, _D), jnp.float32),
        jax.ShapeDtypeStruct((_N, _DE), jnp.float32),
        jax.ShapeDtypeStruct((_N, _DE), jnp.float32),
        jax.ShapeDtypeStruct((_N, _D), jnp.float32),
    ],
)


def _encode_edges_body(ea_ref, wee_ref, bee_ref, we0_ref, be_ref,
                       ef0_ref, t0_ref):
    ef0 = jnp.dot(ea_ref[...], wee_ref[...],
                  preferred_element_type=jnp.float32) + bee_ref[...]
    ef0_ref[...] = ef0
    t0_ref[...] = jnp.dot(ef0, we0_ref[...],
                          preferred_element_type=jnp.float32) + be_ref[...]


_encode_edges = pl.pallas_call(
    _encode_edges_body,
    grid=(_E2 // _BE2,),
    in_specs=[
        pl.BlockSpec((_BE2, 32), lambda i: (i, 0)),
        pl.BlockSpec((32, _D), lambda i: (0, 0)),
        pl.BlockSpec((1, _D), lambda i: (0, 0)),
        pl.BlockSpec((_D, _D), lambda i: (0, 0)),
        pl.BlockSpec((1, _D), lambda i: (0, 0)),
    ],
    out_specs=[
        pl.BlockSpec((_BE2, _D), lambda i: (i, 0)),
        pl.BlockSpec((_BE2, _D), lambda i: (i, 0)),
    ],
    out_shape=[
        jax.ShapeDtypeStruct((_E2, _D), jnp.float32),
        jax.ShapeDtypeStruct((_E2, _D), jnp.float32),
    ],
)


def _proj_body(nf_ref, a0s_ref, a0d_ref, ws1_ref, wd1_ref, as_ref, ad_ref):
    nf = nf_ref[...]
    as_ref[...] = a0s_ref[...] + jnp.dot(nf, ws1_ref[...],
                                         preferred_element_type=jnp.float32)
    ad_ref[...] = a0d_ref[...] + jnp.dot(nf, wd1_ref[...],
                                         preferred_element_type=jnp.float32)


_proj = pl.pallas_call(
    _proj_body,
    grid=(_N // _BN,),
    in_specs=[
        pl.BlockSpec((_BN, _D), lambda i: (i, 0)),
        pl.BlockSpec((_BN, _DE), lambda i: (i, 0)),
        pl.BlockSpec((_BN, _DE), lambda i: (i, 0)),
        pl.BlockSpec((_D, _DE), lambda i: (0, 0)),
        pl.BlockSpec((_D, _DE), lambda i: (0, 0)),
    ],
    out_specs=[
        pl.BlockSpec((_BN, _DE), lambda i: (i, 0)),
        pl.BlockSpec((_BN, _DE), lambda i: (i, 0)),
    ],
    out_shape=[
        jax.ShapeDtypeStruct((_N, _DE), jnp.float32),
        jax.ShapeDtypeStruct((_N, _DE), jnp.float32),
    ],
)


def _edge_t_body(t0_ref, ef_ref, we1_ref, t_ref):
    t_ref[...] = t0_ref[...] + jnp.dot(ef_ref[...], we1_ref[...],
                                       preferred_element_type=jnp.float32)


_edge_t = pl.pallas_call(
    _edge_t_body,
    grid=(_E2 // _BE2,),
    in_specs=[
        pl.BlockSpec((_BE2, _D), lambda i: (i, 0)),
        pl.BlockSpec((_BE2, _D), lambda i: (i, 0)),
        pl.BlockSpec((_D, _D), lambda i: (0, 0)),
    ],
    out_specs=pl.BlockSpec((_BE2, _D), lambda i: (i, 0)),
    out_shape=jax.ShapeDtypeStruct((_E2, _D), jnp.float32),
)


def _node_update_body(cn_ref, nf_ref, p0_ref, p1_ref, wn1_ref, wna_ref,
                      out_ref):
    aggr = p0_ref[...] + p1_ref[...]
    v = (cn_ref[...]
         + jnp.dot(nf_ref[...], wn1_ref[...], preferred_element_type=jnp.float32)
         + jnp.dot(aggr, wna_ref[...], preferred_element_type=jnp.float32))
    out_ref[...] = jnp.maximum(v, 0.0)


_node_update = pl.pallas_call(
    _node_update_body,
    grid=(_N // _BN,),
    in_specs=[
        pl.BlockSpec((_BN, _D), lambda i: (i, 0)),
        pl.BlockSpec((_BN, _D), lambda i: (i, 0)),
        pl.BlockSpec((_BN, _DE), lambda i: (i, 0)),
        pl.BlockSpec((_BN, _DE), lambda i: (i, 0)),
        pl.BlockSpec((_D, _D), lambda i: (0, 0)),
        pl.BlockSpec((_DE, _D), lambda i: (0, 0)),
    ],
    out_specs=pl.BlockSpec((_BN, _D), lambda i: (i, 0)),
    out_shape=jax.ShapeDtypeStruct((_N, _D), jnp.float32),
)


def _head_body(nf_ref, w_ref, b_ref, out_ref):
    out_ref[...] = jnp.dot(nf_ref[...], w_ref[...],
                           preferred_element_type=jnp.float32) + b_ref[...]


def _make_head(k):
    return pl.pallas_call(
        _head_body,
        grid=(_N // _BN,),
        in_specs=[
            pl.BlockSpec((_BN, _D), lambda i: (i, 0)),
            pl.BlockSpec((_D, k), lambda i: (0, 0)),
            pl.BlockSpec((1, k), lambda i: (0, 0)),
        ],
        out_specs=pl.BlockSpec((_BN, k), lambda i: (i, 0)),
        out_shape=jax.ShapeDtypeStruct((_N, k), jnp.float32),
    )


_head1 = _make_head(1)
_head18 = _make_head(18)


def _edge_sc_body(as_hbm, ad_hbm, t_hbm, src_hbm, dst_hbm, ef_out, part_out,
                  sidx, didx, gs, gd, tb, sb, so, stage, accum,
                  semi, semg0, semg1, semt0, semt1, semw0, semw1,
                  semc0, semc1):
    cid = lax.axis_index("c")
    sid = lax.axis_index("s")
    wid = sid * _NC + cid
    semg = (semg0, semg1)
    semt = (semt0, semt1)
    semw = (semw0, semw1)
    semc = (semc0, semc1)

    hi1 = pltpu.async_copy(src_hbm.at[pl.ds(wid * _NCH, _NCH)], sidx, semi)
    hi2 = pltpu.async_copy(dst_hbm.at[pl.ds(wid * _NCH, _NCH)], didx, semi)

    def zrow(r, _):
        for j in range(4):
            stage[r, pl.ds(j * 16, 16)] = jnp.zeros((16,), jnp.float32)
        return 0

    lax.fori_loop(0, _ZR, zrow, 0)
    for k in range(_NPS // _ZR):
        pltpu.sync_copy(stage, accum.at[pl.ds(sid * _NPS + k * _ZR, _ZR)])

    @pl.when(sid == _NS - 1)
    def _():
        pltpu.sync_copy(stage.at[pl.ds(0, _TAIL)],
                        accum.at[pl.ds(_NPS * _NS, _TAIL)])

    hi1.wait()
    hi2.wait()
    plsc.subcore_barrier()

    def issue_in(i, b):
        pltpu.async_copy(as_hbm.at[sidx.at[i]], gs.at[b], semg[b])
        pltpu.async_copy(ad_hbm.at[didx.at[i]], gd.at[b], semg[b])
        pltpu.async_copy(t_hbm.at[pl.ds(wid * _E2PW + i * _CH2, _CH2)],
                         tb.at[b], semt[b])

    def wait_in(i, b):
        pltpu.make_async_copy(as_hbm.at[sidx.at[i]], gs.at[b], semg[b]).wait()
        pltpu.make_async_copy(ad_hbm.at[didx.at[i]], gd.at[b], semg[b]).wait()
        pltpu.make_async_copy(t_hbm.at[pl.ds(wid * _E2PW + i * _CH2, _CH2)],
                              tb.at[b], semt[b]).wait()

    def emit_out(i, b):
        pltpu.async_copy(so.at[b],
                         ef_out.at[pl.ds(wid * _E2PW + i * _CH2, _CH2)],
                         semw[b])
        pltpu.async_copy(sb.at[b], accum.at[didx.at[i]], semc[b], add=True)

    def wait_out(i, b):
        pltpu.make_async_copy(
            so.at[b], ef_out.at[pl.ds(wid * _E2PW + i * _CH2, _CH2)],
            semw[b]).wait()
        pltpu.make_async_copy(sb.at[b], accum.at[didx.at[i]], semc[b]).wait()

    def compute(b):
        def rowf(rr, _):
            for jj in range(8):
                e = 2 * rr + jj // 4
                sl = pl.ds((jj % 4) * 16, 16)
                v = jnp.maximum(
                    gs[b, e, sl] + gd[b, e, sl] + tb[b, rr,
                                                     pl.ds(jj * 16, 16)],
                    0.0)
                sb[b, e, sl] = v
                so[b, rr, pl.ds(jj * 16, 16)] = v
            return 0

        lax.fori_loop(0, _CH2, rowf, 0)

    def step(i, b, t):
        @pl.when(t > 0)
        def _():
            wait_out(i - 2, b)

        wait_in(i, b)
        compute(b)
        emit_out(i, b)

        @pl.when(i + 2 < _NCH)
        def _():
            issue_in(i + 2, b)

    issue_in(0, 0)
    issue_in(1, 1)

    def pair(t, _):
        step(2 * t, 0, t)
        step(2 * t + 1, 1, t)
        return 0

    lax.fori_loop(0, (_NCH - 1) // 2, pair, 0)
    last = _NCH - 1
    wait_out(last - 2, 0)
    wait_in(last, 0)
    compute(0)
    emit_out(last, 0)
    wait_out(last - 1, 1)
    wait_out(last, 0)
    plsc.subcore_barrier()

    for k in range(_NPS // _ZR):
        off = sid * _NPS + k * _ZR
        pltpu.sync_copy(accum.at[pl.ds(off, _ZR)], stage)
        pltpu.sync_copy(stage, part_out.at[cid, pl.ds(off, _ZR)])

    @pl.when(sid == _NS - 1)
    def _():
        off = _NPS * _NS
        pltpu.sync_copy(accum.at[pl.ds(off, _TAIL)], stage.at[pl.ds(0, _TAIL)])
        pltpu.sync_copy(stage.at[pl.ds(0, _TAIL)],
                        part_out.at[cid, pl.ds(off, _TAIL)])


@functools.cache
def _edge_sc():
    return pl.kernel(
        _edge_sc_body,
        mesh=plsc.VectorSubcoreMesh(core_axis_name="c", subcore_axis_name="s"),
        compiler_params=pltpu.CompilerParams(use_tc_tiling_on_sc=False),
        out_type=[
            jax.ShapeDtypeStruct((_E2, _D), jnp.float32),
            jax.ShapeDtypeStruct((_NC, _N, _DE), jnp.float32),
        ],
        scratch_types=[
            pltpu.VMEM((_NCH, _CH), jnp.int32),
            pltpu.VMEM((_NCH, _CH), jnp.int32),
            pltpu.VMEM((2, _CH, _DE), jnp.float32),
            pltpu.VMEM((2, _CH, _DE), jnp.float32),
            pltpu.VMEM((2, _CH2, _D), jnp.float32),
            pltpu.VMEM((2, _CH, _DE), jnp.float32),
            pltpu.VMEM((2, _CH2, _D), jnp.float32),
            pltpu.VMEM((_ZR, _DE), jnp.float32),
            pltpu.VMEM_SHARED((_N, _DE), jnp.float32),
        ] + [pltpu.SemaphoreType.DMA] * 9,
    )


def kernel(x, edge_attr, edge_index, node_types, W_ne, b_ne, W_ee, b_ee,
           W_e, b_e, W_n, b_n, W_tag, b_tag, W_nc, b_nc, W_cls, b_cls):
    del node_types
    src = edge_index[0].astype(jnp.int32).reshape(_NW * _NCH, _CH)
    dst = edge_index[1].astype(jnp.int32).reshape(_NW * _NCH, _CH)

    We_s0, We_s1 = W_e[0:128], W_e[128:256]
    We_d0, We_d1 = W_e[256:384], W_e[384:512]
    We_e0, We_e1 = W_e[512:576], W_e[576:640]
    Wn_0, Wn_1, Wn_a = W_n[0:128], W_n[128:256], W_n[256:320]

    def _bdiag(w):
        z = jnp.zeros_like(w)
        return jnp.concatenate(
            [jnp.concatenate([w, z], axis=1),
             jnp.concatenate([z, w], axis=1)], axis=0)

    def _btile(b):
        return jnp.concatenate([b, b]).reshape(1, -1)

    nf0, A0s, A0d, Cn = _encode_nodes(
        x, W_ne, b_ne.reshape(1, -1), We_s0, We_d0, Wn_0, b_n.reshape(1, -1))
    ef0, T0 = _encode_edges(
        edge_attr.reshape(_E2, 32), _bdiag(W_ee), _btile(b_ee),
        _bdiag(We_e0), _btile(b_e))

    nf, ef = nf0, ef0
    pred_tag = None
    We_e1d = _bdiag(We_e1)
    for layer in range(4):
        As, Ad = _proj(nf, A0s, A0d, We_s1, We_d1)
        T = _edge_t(T0, ef, We_e1d)
        ef, parts = _edge_sc()(As, Ad, T, src, dst)
        nf = _node_update(Cn, nf, parts[0], parts[1], Wn_1, Wn_a)
        if layer == 1:
            pred_tag = _head1(nf, W_tag, b_tag.reshape(1, -1))[:, 0]

    Wh = jnp.concatenate([W_nc, W_cls], axis=1)
    bh = jnp.concatenate([b_nc, b_cls]).reshape(1, -1)
    out = _head18(nf, Wh, bh)
    return out[:, 0], out[:, 1:], pred_tag
